# baseline (speedup 1.0000x reference)
import jax
import jax.numpy as jnp
from jax import lax
from jax.experimental import pallas as pl
from jax.experimental.pallas import tpu as pltpu
from jax.experimental.pallas import tpu_sc as plsc

D = 1024
SCALE = 32.0

NC = 2
NS = 16
NW = NC * NS

B = 4 * 8192
B_PER_W = B // NW
C = 16
N_CHUNKS = B_PER_W // C
NSLOT = 7
LOOKAHEAD = 5


def _body(w_hbm, xi_hbm, out_hbm, idx_v, buf,
          gsem0, gsem1, gsem2, gsem3, gsem4, gsem5, gsem6,
          ssem0, ssem1, ssem2, ssem3, ssem4, ssem5, ssem6):
    wid = lax.axis_index("s") * NC + lax.axis_index("c")
    base = wid * B_PER_W
    pltpu.sync_copy(xi_hbm.at[pl.ds(base, B_PER_W)], idx_v)

    gsems = (gsem0, gsem1, gsem2, gsem3, gsem4, gsem5, gsem6)
    ssems = (ssem0, ssem1, ssem2, ssem3, ssem4, ssem5, ssem6)

    def slot(p):
        return buf.at[pl.ds(p * C, C)]

    def issue_gather(ci, p):
        pltpu.async_copy(w_hbm.at[idx_v.at[pl.ds(ci * C, C)]],
                         slot(p), gsems[p])

    def wait_gather(p):
        pltpu.make_async_copy(w_hbm.at[pl.ds(0, C)], slot(p),
                              gsems[p]).wait()

    def wait_store(p):
        pltpu.make_async_copy(slot(p), out_hbm.at[pl.ds(0, C)],
                              ssems[p]).wait()

    def scale(p):
        def row_body(i, c2):
            row = lax.shift_right_logical(i, 2)
            colb = pl.multiple_of(
                lax.shift_left(lax.bitwise_and(i, 3), 8), 256)
            for j in range(D // 64):
                sl = (row, pl.ds(colb + j * 16, 16))
                slot(p)[sl] = slot(p)[sl] * SCALE
            return c2

        lax.fori_loop(0, 4 * C, row_body, 0)

    def issue_store(ci, p):
        pltpu.async_copy(slot(p), out_hbm.at[pl.ds(base + ci * C, C)],
                         ssems[p])

    for ci in range(LOOKAHEAD):
        issue_gather(ci, ci)

    for ci in range(NSLOT):
        wait_gather(ci)
        scale(ci)
        issue_store(ci, ci)
        pn = (ci + LOOKAHEAD) % NSLOT
        if ci >= NSLOT - LOOKAHEAD:
            wait_store(pn)
        issue_gather(ci + LOOKAHEAD, pn)

    def outer_body(k, carry):
        for p in range(NSLOT):
            ci = k * NSLOT + p
            wait_gather(p)
            scale(p)
            issue_store(ci, p)
            pn = (p + LOOKAHEAD) % NSLOT

            @pl.when(ci + LOOKAHEAD < N_CHUNKS)
            def _():
                wait_store(pn)
                issue_gather(ci + LOOKAHEAD, pn)
        return carry

    lax.fori_loop(1, (N_CHUNKS - 1) // NSLOT, outer_body, 0)

    wait_gather(0)
    scale(0)
    issue_store(N_CHUNKS - 1, 0)

    for p in range(NSLOT):
        wait_store(p)


@jax.jit
def kernel(x, W):
    xflat = x.reshape(-1)
    mesh = plsc.VectorSubcoreMesh(
        core_axis_name="c", subcore_axis_name="s", num_cores=NC, num_subcores=NS
    )
    out = pl.kernel(
        _body,
        out_type=jax.ShapeDtypeStruct((B, D), jnp.float32),
        mesh=mesh,
        scratch_types=[
            pltpu.VMEM((B_PER_W,), jnp.int32),
            pltpu.VMEM((NSLOT * C, D), jnp.float32),
        ] + [pltpu.SemaphoreType.DMA] * (2 * NSLOT),
    )(W, xflat)
    return out.reshape(x.shape[0], x.shape[1], D)


# speedup vs baseline: 1.0162x; 1.0162x over previous
import jax
import jax.numpy as jnp
from jax import lax
from jax.experimental import pallas as pl
from jax.experimental.pallas import tpu as pltpu
from jax.experimental.pallas import tpu_sc as plsc

D = 1024
SCALE = 32.0

NC = 2
NS = 16
NW = NC * NS

B = 4 * 8192
B_PER_W = B // NW
C = 16
N_CHUNKS = B_PER_W // C
NG = 4
NST = 2
N_OUTER = N_CHUNKS // NG


def _body(w_hbm, xi_hbm, out_hbm, idx_v, gbuf, sbuf,
          gsem0, gsem1, gsem2, gsem3, ssem0, ssem1):
    wid = lax.axis_index("s") * NC + lax.axis_index("c")
    base = wid * B_PER_W
    pltpu.sync_copy(xi_hbm.at[pl.ds(base, B_PER_W)], idx_v)

    gsems = (gsem0, gsem1, gsem2, gsem3)
    ssems = (ssem0, ssem1)

    def gslot(b):
        return gbuf.at[pl.ds(b * C, C)]

    def sslot(b):
        return sbuf.at[pl.ds(b * C, C)]

    def issue_gather(ci, b):
        pltpu.async_copy(w_hbm.at[idx_v.at[pl.ds(ci * C, C)]],
                         gslot(b), gsems[b])

    for b in range(NG):
        issue_gather(b, b)

    def outer_body(k, carry):
        for b in range(NG):
            ci = k * NG + b
            s = b % NST
            pltpu.make_async_copy(w_hbm.at[pl.ds(0, C)], gslot(b),
                                  gsems[b]).wait()

            def wait_store():
                pltpu.make_async_copy(sslot(s), out_hbm.at[pl.ds(0, C)],
                                      ssems[s]).wait()

            if b < NST:
                @pl.when(k > 0)
                def _():
                    wait_store()
            else:
                wait_store()

            def row_body(i, c2):
                row = lax.shift_right_logical(i, 3)
                colb = pl.multiple_of(lax.shift_left(lax.bitwise_and(i, 7), 7), 128)
                for j in range(D // 128):
                    sl = (row, pl.ds(colb + j * 16, 16))
                    sslot(s)[sl] = gslot(b)[sl] * SCALE
                return c2

            lax.fori_loop(0, 8 * C, row_body, 0)

            @pl.when(k < N_OUTER - 1)
            def _():
                issue_gather(ci + NG, b)

            pltpu.async_copy(sslot(s), out_hbm.at[pl.ds(base + ci * C, C)],
                             ssems[s])
        return carry

    lax.fori_loop(0, N_OUTER, outer_body, 0)

    for s in range(NST):
        pltpu.make_async_copy(sslot(s), out_hbm.at[pl.ds(0, C)],
                              ssems[s]).wait()


@jax.jit
def kernel(x, W):
    xflat = x.reshape(-1)
    mesh = plsc.VectorSubcoreMesh(
        core_axis_name="c", subcore_axis_name="s", num_cores=NC, num_subcores=NS
    )
    out = pl.kernel(
        _body,
        out_type=jax.ShapeDtypeStruct((B, D), jnp.float32),
        mesh=mesh,
        scratch_types=[
            pltpu.VMEM((B_PER_W,), jnp.int32),
            pltpu.VMEM((NG * C, D), jnp.float32),
            pltpu.VMEM((NST * C, D), jnp.float32),
            pltpu.SemaphoreType.DMA,
            pltpu.SemaphoreType.DMA,
            pltpu.SemaphoreType.DMA,
            pltpu.SemaphoreType.DMA,
            pltpu.SemaphoreType.DMA,
            pltpu.SemaphoreType.DMA,
        ],
    )(W, xflat)
    return out.reshape(x.shape[0], x.shape[1], D)
